# baseline (speedup 1.0000x reference)
import functools

import jax
import jax.numpy as jnp
from jax import lax
from jax.experimental import pallas as pl
from jax.experimental.pallas import tpu as pltpu
from jax.experimental.pallas import tpu_sc as plsc

NC = 2
NS = 16
NW = NC * NS
L = 16
CHUNK = 128


def _build(B, D):
    assert D == L
    assert B % (NW * CHUNK) == 0
    bpw = B // NW
    nchunk = bpw // CHUNK
    ngroup = bpw // L

    mesh = plsc.VectorSubcoreMesh(
        core_axis_name="c", subcore_axis_name="s",
        num_cores=NC, num_subcores=NS)

    @functools.partial(
        pl.kernel,
        out_type=jax.ShapeDtypeStruct((B,), jnp.float32),
        mesh=mesh,
        compiler_params=pltpu.CompilerParams(use_tc_tiling_on_sc=False,
                                             needs_layout_passes=False),
        scratch_types=[
            pltpu.VMEM((nchunk, CHUNK), jnp.int32),
            pltpu.VMEM((nchunk, CHUNK), jnp.int32),
            pltpu.VMEM((bpw, D), jnp.float32),
            pltpu.VMEM((bpw, D), jnp.float32),
            pltpu.VMEM((bpw,), jnp.float32),
            pltpu.VMEM((bpw,), jnp.float32),
            pltpu.VMEM((bpw,), jnp.float32),
            pltpu.VMEM((L,), jnp.float32),
            pltpu.SemaphoreType.DMA,
        ],
    )
    def svdpp(uid_hbm, iid_hbm, p_hbm, q_hbm, bu_hbm, bi_hbm, mu_hbm,
              out_hbm, uid_v, iid_v, prow, qrow, bu_v, bi_v, out_v,
              mu_v, sem):
        wid = lax.axis_index("s") * NC + lax.axis_index("c")
        base = wid * bpw

        pltpu.sync_copy(mu_hbm, mu_v)
        for j in range(nchunk):
            pltpu.sync_copy(uid_hbm.at[pl.ds(base + j * CHUNK, CHUNK)],
                            uid_v.at[j])
            pltpu.sync_copy(iid_hbm.at[pl.ds(base + j * CHUNK, CHUNK)],
                            iid_v.at[j])

        copies = []
        for j in range(nchunk):
            sl = pl.ds(j * CHUNK, CHUNK)
            copies.append(pltpu.async_copy(p_hbm.at[uid_v.at[j]],
                                           prow.at[sl], sem))
            copies.append(pltpu.async_copy(q_hbm.at[iid_v.at[j]],
                                           qrow.at[sl], sem))
            copies.append(pltpu.async_copy(bu_hbm.at[uid_v.at[j]],
                                           bu_v.at[sl], sem))
            copies.append(pltpu.async_copy(bi_hbm.at[iid_v.at[j]],
                                           bi_v.at[sl], sem))
        for c in copies:
            c.wait()

        mu_vec = mu_v[...]

        def group(g, carry):
            rows = g * L + lax.iota(jnp.int32, L)
            acc = bu_v[pl.ds(g * L, L)] + bi_v[pl.ds(g * L, L)] + mu_vec
            for d in range(D):
                dcol = jnp.full((L,), d, jnp.int32)
                acc = acc + (plsc.load_gather(prow, [rows, dcol])
                             * plsc.load_gather(qrow, [rows, dcol]))
            out_v[pl.ds(g * L, L)] = acc
            return carry

        lax.fori_loop(0, ngroup, group, 0)
        pltpu.sync_copy(out_v, out_hbm.at[pl.ds(base, bpw)])

    return svdpp


def kernel(user_id, item_id, u_i_dict, P, Q, user_bias, item_bias,
           global_bias):
    del u_i_dict
    B = user_id.shape[0]
    D = P.shape[1]
    mu16 = jnp.broadcast_to(global_bias.astype(jnp.float32), (L,))
    fn = _build(B, D)
    return fn(user_id.astype(jnp.int32), item_id.astype(jnp.int32),
              P, Q, user_bias.reshape(-1), item_bias.reshape(-1), mu16)


# speedup vs baseline: 3.1726x; 3.1726x over previous
import functools

import jax
import jax.numpy as jnp
from jax import lax
from jax.experimental import pallas as pl
from jax.experimental.pallas import tpu as pltpu
from jax.experimental.pallas import tpu_sc as plsc

NC = 2
NS = 16
NW = NC * NS
L = 16
TCOL = 128
CHUNK = 128
FIRE = 8


def _build(B, D):
    assert D == L
    assert B % (NW * CHUNK) == 0
    bpw = B // NW
    nchunk = bpw // CHUNK
    nbatch = bpw // FIRE

    mesh = plsc.VectorSubcoreMesh(
        core_axis_name="c", subcore_axis_name="s",
        num_cores=NC, num_subcores=NS)

    @functools.partial(
        pl.kernel,
        out_type=jax.ShapeDtypeStruct((B,), jnp.float32),
        mesh=mesh,
        compiler_params=pltpu.CompilerParams(use_tc_tiling_on_sc=True,
                                             needs_layout_passes=False),
        scratch_types=(
            [pltpu.VMEM((nchunk, CHUNK), jnp.int32),
             pltpu.VMEM((nchunk, CHUNK), jnp.int32),
             pltpu.VMEM((bpw,), jnp.float32),
             pltpu.VMEM((bpw,), jnp.float32),
             pltpu.VMEM((bpw,), jnp.float32),
             pltpu.VMEM((L,), jnp.float32)]
            + [pltpu.VMEM((D, TCOL), jnp.float32) for _ in range(2 * FIRE)]
            + [pltpu.VMEM((bpw * D,), jnp.float32),
               pltpu.VMEM((bpw * D,), jnp.float32)]
            + [pltpu.SemaphoreType.DMA, pltpu.SemaphoreType.DMA]
        ),
    )
    def svdpp(uid_hbm, iid_hbm, pt_hbm, qt_hbm, bu_hbm, bi_hbm, mu_hbm,
              out_hbm, uid_v, iid_v, bu_v, bi_v, out_v, mu_v, *rest):
        bufs = rest[:2 * FIRE]
        prow, qrow, sem, semb = rest[2 * FIRE:]
        wid = lax.axis_index("s") * NC + lax.axis_index("c")
        base = wid * bpw

        pltpu.sync_copy(mu_hbm, mu_v)
        for j in range(nchunk):
            pltpu.sync_copy(uid_hbm.at[pl.ds(base + j * CHUNK, CHUNK)],
                            uid_v.at[j])
            pltpu.sync_copy(iid_hbm.at[pl.ds(base + j * CHUNK, CHUNK)],
                            iid_v.at[j])

        bcopies = []
        for j in range(nchunk):
            sl = pl.ds(j * CHUNK, CHUNK)
            bcopies.append(pltpu.async_copy(bu_hbm.at[uid_v.at[j]],
                                            bu_v.at[sl], semb))
            bcopies.append(pltpu.async_copy(bi_hbm.at[iid_v.at[j]],
                                            bi_v.at[sl], semb))
        for c in bcopies:
            c.wait()

        mu_vec = mu_v[...]
        lane = lax.iota(jnp.int32, L)

        def batch(v, carry):
            j = v // (CHUNK // L)
            off = (v % (CHUNK // L)) * L
            uvec = uid_v[j, pl.ds(off, L)]
            ivec = iid_v[j, pl.ds(off, L)]
            for half in range(L // FIRE):
                cs = []
                for f in range(FIRE):
                    f0 = half * FIRE + f
                    cu = pl.multiple_of((uvec[f0] // TCOL) * TCOL, TCOL)
                    ci = pl.multiple_of((ivec[f0] // TCOL) * TCOL, TCOL)
                    cs.append(pltpu.async_copy(pt_hbm.at[:, pl.ds(cu, TCOL)],
                                               bufs[2 * f], sem))
                    cs.append(pltpu.async_copy(qt_hbm.at[:, pl.ds(ci, TCOL)],
                                               bufs[2 * f + 1], sem))
                for c in cs:
                    c.wait()
                for f in range(FIRE):
                    f0 = half * FIRE + f
                    ru = jnp.full((L,), uvec[f0] % TCOL, jnp.int32)
                    ri = jnp.full((L,), ivec[f0] % TCOL, jnp.int32)
                    pv = plsc.load_gather(bufs[2 * f], [lane, ru])
                    qv = plsc.load_gather(bufs[2 * f + 1], [lane, ri])
                    prow[pl.ds((v * L + f0) * D, D)] = pv
                    qrow[pl.ds((v * L + f0) * D, D)] = qv
            return carry

        lax.fori_loop(0, bpw // L, batch, 0)

        def group(g, carry):
            flat = (g * L + lane) * D
            acc = bu_v[pl.ds(g * L, L)] + bi_v[pl.ds(g * L, L)] + mu_vec
            for d in range(D):
                acc = acc + (plsc.load_gather(prow, [flat + d])
                             * plsc.load_gather(qrow, [flat + d]))
            out_v[pl.ds(g * L, L)] = acc
            return carry

        lax.fori_loop(0, bpw // L, group, 0)
        pltpu.sync_copy(out_v, out_hbm.at[pl.ds(base, bpw)])

    return svdpp


def kernel(user_id, item_id, u_i_dict, P, Q, user_bias, item_bias,
           global_bias):
    del u_i_dict
    B = user_id.shape[0]
    D = P.shape[1]
    mu16 = jnp.broadcast_to(global_bias.astype(jnp.float32), (L,))
    fn = _build(B, D)
    return fn(user_id.astype(jnp.int32), item_id.astype(jnp.int32),
              P.T, Q.T, user_bias.reshape(-1), item_bias.reshape(-1), mu16)


# speedup vs baseline: 3.5068x; 1.1053x over previous
import functools

import jax
import jax.numpy as jnp
from jax import lax
from jax.experimental import pallas as pl
from jax.experimental.pallas import tpu as pltpu
from jax.experimental.pallas import tpu_sc as plsc

NC = 2
NS = 16
NW = NC * NS
L = 16
TCOL = 128
CHUNK = 128
FIRE = 8


def _build(B, D):
    assert D == L
    assert B % (NW * CHUNK) == 0
    bpw = B // NW
    nchunk = bpw // CHUNK
    nbatch = bpw // FIRE

    mesh = plsc.VectorSubcoreMesh(
        core_axis_name="c", subcore_axis_name="s",
        num_cores=NC, num_subcores=NS)

    @functools.partial(
        pl.kernel,
        out_type=jax.ShapeDtypeStruct((B,), jnp.float32),
        mesh=mesh,
        compiler_params=pltpu.CompilerParams(use_tc_tiling_on_sc=True,
                                             needs_layout_passes=False),
        scratch_types=(
            [pltpu.VMEM((nchunk, CHUNK), jnp.int32),
             pltpu.VMEM((nchunk, CHUNK), jnp.int32),
             pltpu.VMEM((bpw,), jnp.float32),
             pltpu.VMEM((bpw,), jnp.float32),
             pltpu.VMEM((bpw,), jnp.float32),
             pltpu.VMEM((L,), jnp.float32)]
            + [pltpu.VMEM((D, TCOL), jnp.float32) for _ in range(4 * FIRE)]
            + [pltpu.VMEM((bpw * D,), jnp.float32),
               pltpu.VMEM((bpw * D,), jnp.float32)]
            + [pltpu.SemaphoreType.DMA, pltpu.SemaphoreType.DMA,
               pltpu.SemaphoreType.DMA]
        ),
    )
    def svdpp(uid_hbm, iid_hbm, pt_hbm, qt_hbm, bu_hbm, bi_hbm, mu_hbm,
              out_hbm, uid_v, iid_v, bu_v, bi_v, out_v, mu_v, *rest):
        set0 = rest[:2 * FIRE]
        set1 = rest[2 * FIRE:4 * FIRE]
        prow, qrow, semA, semB, semb = rest[4 * FIRE:]
        wid = lax.axis_index("s") * NC + lax.axis_index("c")
        base = wid * bpw

        pltpu.sync_copy(mu_hbm, mu_v)
        for j in range(nchunk):
            pltpu.sync_copy(uid_hbm.at[pl.ds(base + j * CHUNK, CHUNK)],
                            uid_v.at[j])
            pltpu.sync_copy(iid_hbm.at[pl.ds(base + j * CHUNK, CHUNK)],
                            iid_v.at[j])

        bcopies = []
        for j in range(nchunk):
            sl = pl.ds(j * CHUNK, CHUNK)
            bcopies.append(pltpu.async_copy(bu_hbm.at[uid_v.at[j]],
                                            bu_v.at[sl], semb))
            bcopies.append(pltpu.async_copy(bi_hbm.at[iid_v.at[j]],
                                            bi_v.at[sl], semb))
        for c in bcopies:
            c.wait()

        mu_vec = mu_v[...]
        lane = lax.iota(jnp.int32, L)

        def loadvec(v):
            j = v // (CHUNK // L)
            off = (v % (CHUNK // L)) * L
            return uid_v[j, pl.ds(off, L)], iid_v[j, pl.ds(off, L)]

        def fire(bset, sm, uvec, ivec, half):
            for f in range(FIRE):
                f0 = half * FIRE + f
                cu = pl.multiple_of((uvec[f0] // TCOL) * TCOL, TCOL)
                ci = pl.multiple_of((ivec[f0] // TCOL) * TCOL, TCOL)
                pltpu.async_copy(pt_hbm.at[:, pl.ds(cu, TCOL)],
                                 bset[2 * f], sm)
                pltpu.async_copy(qt_hbm.at[:, pl.ds(ci, TCOL)],
                                 bset[2 * f + 1], sm)

        def drain(bset, sm):
            for f in range(2 * FIRE):
                pltpu.make_async_copy(pt_hbm.at[:, pl.ds(0, TCOL)],
                                      bset[f], sm).wait()

        def extract(bset, uvec, ivec, half, v):
            for f in range(FIRE):
                f0 = half * FIRE + f
                ru = jnp.full((L,), uvec[f0] % TCOL, jnp.int32)
                ri = jnp.full((L,), ivec[f0] % TCOL, jnp.int32)
                pv = plsc.load_gather(bset[2 * f], [lane, ru])
                qv = plsc.load_gather(bset[2 * f + 1], [lane, ri])
                prow[pl.ds((v * L + f0) * D, D)] = pv
                qrow[pl.ds((v * L + f0) * D, D)] = qv

        nvec = bpw // L
        u0, i0 = loadvec(0)
        fire(set0, semA, u0, i0, 0)

        def batch(v, carry):
            ucur, icur = carry
            fire(set1, semB, ucur, icur, 1)
            drain(set0, semA)
            extract(set0, ucur, icur, 0, v)
            unext, inext = loadvec(v + 1)
            fire(set0, semA, unext, inext, 0)
            drain(set1, semB)
            extract(set1, ucur, icur, 1, v)
            return (unext, inext)

        ulast, ilast = lax.fori_loop(0, nvec - 1, batch, (u0, i0))
        fire(set1, semB, ulast, ilast, 1)
        drain(set0, semA)
        extract(set0, ulast, ilast, 0, nvec - 1)
        drain(set1, semB)
        extract(set1, ulast, ilast, 1, nvec - 1)

        def group(g, carry):
            flat = (g * L + lane) * D
            acc = bu_v[pl.ds(g * L, L)] + bi_v[pl.ds(g * L, L)] + mu_vec
            for d in range(D):
                acc = acc + (plsc.load_gather(prow, [flat + d])
                             * plsc.load_gather(qrow, [flat + d]))
            out_v[pl.ds(g * L, L)] = acc
            return carry

        lax.fori_loop(0, bpw // L, group, 0)
        pltpu.sync_copy(out_v, out_hbm.at[pl.ds(base, bpw)])

    return svdpp


def kernel(user_id, item_id, u_i_dict, P, Q, user_bias, item_bias,
           global_bias):
    del u_i_dict
    B = user_id.shape[0]
    D = P.shape[1]
    mu16 = jnp.broadcast_to(global_bias.astype(jnp.float32), (L,))
    fn = _build(B, D)
    return fn(user_id.astype(jnp.int32), item_id.astype(jnp.int32),
              P.T, Q.T, user_bias.reshape(-1), item_bias.reshape(-1), mu16)
